# baseline (speedup 1.0000x reference)
import numpy as np
import jax
import jax.numpy as jnp
from jax.experimental import pallas as pl
from jax.experimental.pallas import tpu as pltpu

E = 768
H = 12
DH = 64
BS = 16
NG = 2
NW = 3
NR = 3
S = 4096
B = 2
NB = S // BS
KB = NG + NW + NR
ROWS = 1024
NSC = S // ROWS
G = 4
CH = NB // G
NU = NG + (G + 2) + NR * G
UC = NU * BS


def _block_pattern():
    rng = np.random.default_rng(0)
    half = NW // 2
    offsets = np.arange(NW) - half
    rows = []
    for i in range(NB):
        g = np.arange(NG)
        w = np.clip(i + offsets, 0, NB - 1)
        r = rng.integers(0, NB, size=NR)
        rows.append(np.concatenate([g, w, r]))
    idx = np.stack(rows).astype(np.int32)
    dup = (idx[:, :, None] == idx[:, None, :]) & np.tril(
        np.ones((KB, KB), dtype=bool), -1)[None]
    is_dup = dup.any(-1)
    return idx, is_dup


def _union_pattern(idx, is_dup):
    slots = np.zeros((CH, NU), np.int32)
    mask = np.full((CH, G * BS, UC), -1e9, np.float32)
    for c in range(CH):
        wb = np.clip(c * G - 1 + np.arange(G + 2), 0, NB - 1)
        slots[c] = np.concatenate(
            [np.arange(NG), wb, idx[c * G:(c + 1) * G, NG + NW:].reshape(-1)])
        for r in range(G):
            n = c * G + r
            for j in range(KB):
                if is_dup[n, j]:
                    continue
                v = idx[n, j]
                if j < NG:
                    u = j
                elif j < NG + NW:
                    u = NG + int(np.nonzero(wb == v)[0][0])
                else:
                    u = NG + (G + 2) + r * NR + (j - NG - NW)
                mask[c, r * BS:(r + 1) * BS, u * BS:(u + 1) * BS] = 0.0
    return slots.reshape(-1), mask


_IDX_NP, _ISDUP_NP = _block_pattern()
_SLOTS_NP, _MASK_NP = _union_pattern(_IDX_NP, _ISDUP_NP)


def _proj_kernel(q_ref, k_ref, v_ref, qw_ref, kw_ref, vw_ref, b_ref,
                 qo_ref, ko_ref, vo_ref):
    bias = b_ref[...]
    for x_ref, w_ref, o_ref, i in (
            (q_ref, qw_ref, qo_ref, 0),
            (k_ref, kw_ref, ko_ref, 1),
            (v_ref, vw_ref, vo_ref, 2)):
        x = x_ref[:, 0, 0, :]
        r = jnp.dot(x, w_ref[...], preferred_element_type=jnp.float32)
        r = r + bias[0:1, i * E:(i + 1) * E]
        o_ref[0] = r.astype(o_ref.dtype)


def _attn_kernel(slots_ref, q_ref, k_ref, v_ref, mask_ref, owt_ref, ob_ref,
                 out_ref, kg_ref, vg_ref):
    c = pl.program_id(0)
    msk = mask_ref[0].astype(jnp.float32)
    for b in range(B):
        for u in range(NU):
            src = slots_ref[c * NU + u] * BS
            kg_ref[b, u * BS:(u + 1) * BS, :] = k_ref[b, pl.ds(src, BS), :]
            vg_ref[b, u * BS:(u + 1) * BS, :] = v_ref[b, pl.ds(src, BS), :]
    for b in range(B):
        kgt = jnp.transpose(kg_ref[b], (1, 0))
        q = q_ref[b]
        avs = []
        recips = []
        for h in range(H):
            sl = slice(h * DH, (h + 1) * DH)
            s = jnp.dot(q[:, sl], kgt[sl, :],
                        preferred_element_type=jnp.float32)
            e = jnp.exp(s + msk)
            avs.append(jnp.dot(e.astype(jnp.bfloat16), vg_ref[b, :, sl],
                               preferred_element_type=jnp.float32))
            recips.append(1.0 / jnp.sum(e, axis=-1, keepdims=True))
        oc = jnp.concatenate(
            [av * r for av, r in zip(avs, recips)], axis=1)
        po = jnp.dot(oc.astype(jnp.bfloat16), owt_ref[...],
                     preferred_element_type=jnp.float32)
        out_ref[:, b, 0, :] = po + ob_ref[...]


def kernel(query, key, value, q_w, k_w, v_w, q_b, k_b, v_b, out_w, out_b):
    scale = 1.0 / np.sqrt(np.float32(DH))
    qwt = jnp.transpose(q_w, (1, 0, 2)).reshape(E, E) * scale
    kwt = jnp.transpose(k_w, (1, 0, 2)).reshape(E, E)
    vwt = jnp.transpose(v_w, (1, 0, 2)).reshape(E, E)
    bias = jnp.concatenate(
        [q_b.reshape(1, E) * scale, k_b.reshape(1, E), v_b.reshape(1, E)],
        axis=1)
    owt = out_w.T.astype(jnp.bfloat16)
    obr = out_b.reshape(1, E)

    q4 = query.reshape(S, B, 1, E)
    k4 = key.reshape(S, B, 1, E)
    v4 = value.reshape(S, B, 1, E)

    Q, K, V = pl.pallas_call(
        _proj_kernel,
        grid=(B, NSC),
        in_specs=[
            pl.BlockSpec((ROWS, 1, 1, E), lambda b, s: (s, b, 0, 0)),
            pl.BlockSpec((ROWS, 1, 1, E), lambda b, s: (s, b, 0, 0)),
            pl.BlockSpec((ROWS, 1, 1, E), lambda b, s: (s, b, 0, 0)),
            pl.BlockSpec((E, E), lambda b, s: (0, 0)),
            pl.BlockSpec((E, E), lambda b, s: (0, 0)),
            pl.BlockSpec((E, E), lambda b, s: (0, 0)),
            pl.BlockSpec((1, 3 * E), lambda b, s: (0, 0)),
        ],
        out_specs=[
            pl.BlockSpec((1, ROWS, E), lambda b, s: (b, s, 0)),
            pl.BlockSpec((1, ROWS, E), lambda b, s: (b, s, 0)),
            pl.BlockSpec((1, ROWS, E), lambda b, s: (b, s, 0)),
        ],
        out_shape=[jax.ShapeDtypeStruct((B, S, E), jnp.bfloat16)] * 3,
    )(q4, k4, v4, qwt, kwt, vwt, bias)

    slots = jnp.asarray(_SLOTS_NP)
    mask = jnp.asarray(_MASK_NP).astype(jnp.bfloat16)

    p4 = pl.pallas_call(
        _attn_kernel,
        grid=(CH,),
        in_specs=[
            pl.BlockSpec(memory_space=pltpu.SMEM),
            pl.BlockSpec((B, G * BS, E), lambda c: (0, c, 0)),
            pl.BlockSpec((B, S, E), lambda c: (0, 0, 0)),
            pl.BlockSpec((B, S, E), lambda c: (0, 0, 0)),
            pl.BlockSpec((1, G * BS, UC), lambda c: (c, 0, 0)),
            pl.BlockSpec((E, E), lambda c: (0, 0)),
            pl.BlockSpec((1, E), lambda c: (0, 0)),
        ],
        out_specs=pl.BlockSpec((G * BS, B, 1, E), lambda c: (c, 0, 0, 0)),
        out_shape=jax.ShapeDtypeStruct((S, B, 1, E), jnp.float32),
        scratch_shapes=[
            pltpu.VMEM((B, UC, E), jnp.bfloat16),
            pltpu.VMEM((B, UC, E), jnp.bfloat16),
        ],
    )(slots, Q, K, V, mask, owt, obr)

    return p4.reshape(S, B, E)


# speedup vs baseline: 1.4171x; 1.4171x over previous
import numpy as np
import jax
import jax.numpy as jnp
from jax.experimental import pallas as pl
from jax.experimental.pallas import tpu as pltpu

E = 768
H = 12
DH = 64
BS = 16
NG = 2
NW = 3
NR = 3
S = 4096
B = 2
NB = S // BS
KB = NG + NW + NR
ROWS = 1024
NSC = S // ROWS
G = 16
CH = NB // G
NU = NG + (G + 2) + NR * G
UC = NU * BS


def _block_pattern():
    rng = np.random.default_rng(0)
    half = NW // 2
    offsets = np.arange(NW) - half
    rows = []
    for i in range(NB):
        g = np.arange(NG)
        w = np.clip(i + offsets, 0, NB - 1)
        r = rng.integers(0, NB, size=NR)
        rows.append(np.concatenate([g, w, r]))
    idx = np.stack(rows).astype(np.int32)
    dup = (idx[:, :, None] == idx[:, None, :]) & np.tril(
        np.ones((KB, KB), dtype=bool), -1)[None]
    is_dup = dup.any(-1)
    return idx, is_dup


def _union_pattern(idx, is_dup):
    slots = np.zeros((CH, NU), np.int32)
    mask = np.full((CH, G * BS, UC), -1e9, np.float32)
    for c in range(CH):
        wb = np.clip(c * G - 1 + np.arange(G + 2), 0, NB - 1)
        slots[c] = np.concatenate(
            [np.arange(NG), wb, idx[c * G:(c + 1) * G, NG + NW:].reshape(-1)])
        for r in range(G):
            n = c * G + r
            for j in range(KB):
                if is_dup[n, j]:
                    continue
                v = idx[n, j]
                if j < NG:
                    u = j
                elif j < NG + NW:
                    u = NG + int(np.nonzero(wb == v)[0][0])
                else:
                    u = NG + (G + 2) + r * NR + (j - NG - NW)
                mask[c, r * BS:(r + 1) * BS, u * BS:(u + 1) * BS] = 0.0
    return slots.reshape(-1), mask


_IDX_NP, _ISDUP_NP = _block_pattern()
_SLOTS_NP, _MASK_NP = _union_pattern(_IDX_NP, _ISDUP_NP)


def _proj_kernel(q_ref, k_ref, v_ref, qw_ref, kw_ref, vw_ref, b_ref,
                 qo_ref, ko_ref, vo_ref):
    bias = b_ref[...]
    for x_ref, w_ref, o_ref, i in (
            (q_ref, qw_ref, qo_ref, 0),
            (k_ref, kw_ref, ko_ref, 1),
            (v_ref, vw_ref, vo_ref, 2)):
        x = x_ref[:, 0, 0, :]
        r = jnp.dot(x, w_ref[...], preferred_element_type=jnp.float32)
        r = r + bias[0:1, i * E:(i + 1) * E]
        o_ref[0] = r.astype(o_ref.dtype)


def _attn_kernel(slots_ref, q_ref, k_ref, v_ref, mask_ref, owt_ref, ob_ref,
                 out_ref, kg_ref, vg_ref):
    c = pl.program_id(0)
    msk = mask_ref[0].astype(jnp.float32)
    for b in range(B):
        for u in range(NU):
            src = slots_ref[c * NU + u] * BS
            kg_ref[b, u * BS:(u + 1) * BS, :] = k_ref[b, pl.ds(src, BS), :]
            vg_ref[b, u * BS:(u + 1) * BS, :] = v_ref[b, pl.ds(src, BS), :]
    for b in range(B):
        kgt = jnp.transpose(kg_ref[b], (1, 0))
        q = q_ref[b]
        avs = []
        recips = []
        for h in range(H):
            sl = slice(h * DH, (h + 1) * DH)
            s = jnp.dot(q[:, sl], kgt[sl, :],
                        preferred_element_type=jnp.float32)
            e = jnp.exp(s + msk)
            avs.append(jnp.dot(e.astype(jnp.bfloat16), vg_ref[b, :, sl],
                               preferred_element_type=jnp.float32))
            recips.append(1.0 / jnp.sum(e, axis=-1, keepdims=True))
        oc = jnp.concatenate(
            [av * r for av, r in zip(avs, recips)], axis=1)
        po = jnp.dot(oc.astype(jnp.bfloat16), owt_ref[...],
                     preferred_element_type=jnp.float32)
        out_ref[:, b, 0, :] = po + ob_ref[...]


def kernel(query, key, value, q_w, k_w, v_w, q_b, k_b, v_b, out_w, out_b):
    scale = 1.0 / np.sqrt(np.float32(DH))
    qwt = jnp.transpose(q_w, (1, 0, 2)).reshape(E, E) * scale
    kwt = jnp.transpose(k_w, (1, 0, 2)).reshape(E, E)
    vwt = jnp.transpose(v_w, (1, 0, 2)).reshape(E, E)
    bias = jnp.concatenate(
        [q_b.reshape(1, E) * scale, k_b.reshape(1, E), v_b.reshape(1, E)],
        axis=1)
    owt = out_w.T.astype(jnp.bfloat16)
    obr = out_b.reshape(1, E)

    q4 = query.reshape(S, B, 1, E)
    k4 = key.reshape(S, B, 1, E)
    v4 = value.reshape(S, B, 1, E)

    Q, K, V = pl.pallas_call(
        _proj_kernel,
        grid=(B, NSC),
        in_specs=[
            pl.BlockSpec((ROWS, 1, 1, E), lambda b, s: (s, b, 0, 0)),
            pl.BlockSpec((ROWS, 1, 1, E), lambda b, s: (s, b, 0, 0)),
            pl.BlockSpec((ROWS, 1, 1, E), lambda b, s: (s, b, 0, 0)),
            pl.BlockSpec((E, E), lambda b, s: (0, 0)),
            pl.BlockSpec((E, E), lambda b, s: (0, 0)),
            pl.BlockSpec((E, E), lambda b, s: (0, 0)),
            pl.BlockSpec((1, 3 * E), lambda b, s: (0, 0)),
        ],
        out_specs=[
            pl.BlockSpec((1, ROWS, E), lambda b, s: (b, s, 0)),
            pl.BlockSpec((1, ROWS, E), lambda b, s: (b, s, 0)),
            pl.BlockSpec((1, ROWS, E), lambda b, s: (b, s, 0)),
        ],
        out_shape=[jax.ShapeDtypeStruct((B, S, E), jnp.bfloat16)] * 3,
    )(q4, k4, v4, qwt, kwt, vwt, bias)

    slots = jnp.asarray(_SLOTS_NP)
    mask = jnp.asarray(_MASK_NP).astype(jnp.bfloat16)

    p4 = pl.pallas_call(
        _attn_kernel,
        grid=(CH,),
        in_specs=[
            pl.BlockSpec(memory_space=pltpu.SMEM),
            pl.BlockSpec((B, G * BS, E), lambda c: (0, c, 0)),
            pl.BlockSpec((B, S, E), lambda c: (0, 0, 0)),
            pl.BlockSpec((B, S, E), lambda c: (0, 0, 0)),
            pl.BlockSpec((1, G * BS, UC), lambda c: (c, 0, 0)),
            pl.BlockSpec((E, E), lambda c: (0, 0)),
            pl.BlockSpec((1, E), lambda c: (0, 0)),
        ],
        out_specs=pl.BlockSpec((G * BS, B, 1, E), lambda c: (c, 0, 0, 0)),
        out_shape=jax.ShapeDtypeStruct((S, B, 1, E), jnp.float32),
        scratch_shapes=[
            pltpu.VMEM((B, UC, E), jnp.bfloat16),
            pltpu.VMEM((B, UC, E), jnp.bfloat16),
        ],
    )(slots, Q, K, V, mask, owt, obr)

    return p4.reshape(S, B, E)
